# baseline (speedup 1.0000x reference)
import functools

import jax
import jax.numpy as jnp
from jax import lax
from jax.experimental import pallas as pl
from jax.experimental.pallas import tpu as pltpu
from jax.experimental.pallas import tpu_sc as plsc

B, S, D_MODEL = 4, 2048, 4096
ROUTER = 1024
NUM_EXPERTS = 64
TOP_K = 8
THRESHOLD = 0.1
LATERAL = 0.1
NVREG = NUM_EXPERTS // 16
BIG = 3.0e38

DCHUNK = 1024
SCHUNK = 1024
NSTEPS = D_MODEL // DCHUNK
NSSTEPS = S // SCHUNK


def _tc_body(hid_ref, w1_ref, b1_ref, g_ref, be_ref, w2_ref, b2_ref,
             rs_ref, feat_ref, lg_ref, h_ref, pool_ref):
    i = pl.program_id(0)
    j = pl.program_id(1)

    part = jnp.sum(hid_ref[...], axis=1)

    @pl.when(j == 0)
    def _pinit():
        pool_ref[...] = part

    @pl.when(j > 0)
    def _pacc():
        pool_ref[...] += part

    @pl.when(j == NSSTEPS - 1)
    def _proj():
        pooled_j = pool_ref[...] * (1.0 / S)
        hj = jax.lax.dot_general(
            pooled_j, w1_ref[...], (((1,), (1,)), ((), ())),
            preferred_element_type=jnp.float32)

        @pl.when(i == 0)
        def _init():
            h_ref[...] = hj

        @pl.when(i > 0)
        def _acc():
            h_ref[...] += hj

    @pl.when((i == NSTEPS - 1) & (j == NSSTEPS - 1))
    def _head():
        h = h_ref[...] + b1_ref[...]
        mu = jnp.mean(h, axis=-1, keepdims=True)
        var = jnp.mean((h - mu) ** 2, axis=-1, keepdims=True)
        h = (h - mu) / jnp.sqrt(var + 1e-5) * g_ref[...] + be_ref[...]
        h = 0.5 * h * (1.0 + jax.lax.erf(h * (2.0 ** -0.5)))
        features = jax.lax.dot_general(
            h, w2_ref[...], (((1,), (1,)), ((), ())),
            preferred_element_type=jnp.float32) + b2_ref[...]
        feat_ref[...] = features

        affinity = jax.lax.dot_general(
            features, w2_ref[0:NUM_EXPERTS, :], (((1,), (1,)), ((), ())),
            preferred_element_type=jnp.float32)
        lg_ref[...] = rs_ref[...] + 0.1 * affinity


def _tc_dense(hidden_states, W1, b1, gamma, beta, W2, b2, routing_scores):
    return pl.pallas_call(
        _tc_body,
        grid=(NSTEPS, NSSTEPS),
        in_specs=[
            pl.BlockSpec((B, SCHUNK, DCHUNK), lambda i, j: (0, j, i)),
            pl.BlockSpec((ROUTER, DCHUNK), lambda i, j: (0, i)),
            pl.BlockSpec((1, ROUTER), lambda i, j: (0, 0)),
            pl.BlockSpec((1, ROUTER), lambda i, j: (0, 0)),
            pl.BlockSpec((1, ROUTER), lambda i, j: (0, 0)),
            pl.BlockSpec((ROUTER, ROUTER), lambda i, j: (0, 0)),
            pl.BlockSpec((1, ROUTER), lambda i, j: (0, 0)),
            pl.BlockSpec((1, NUM_EXPERTS), lambda i, j: (0, 0)),
        ],
        out_specs=[
            pl.BlockSpec((B, ROUTER), lambda i, j: (0, 0)),
            pl.BlockSpec((B, NUM_EXPERTS), lambda i, j: (0, 0)),
        ],
        out_shape=[
            jax.ShapeDtypeStruct((B, ROUTER), jnp.float32),
            jax.ShapeDtypeStruct((B, NUM_EXPERTS), jnp.float32),
        ],
        scratch_shapes=[pltpu.VMEM((B, ROUTER), jnp.float32),
                        pltpu.VMEM((B, DCHUNK), jnp.float32)],
        compiler_params=pltpu.CompilerParams(
            dimension_semantics=("arbitrary", "arbitrary")),
    )(hidden_states, W1, b1.reshape(1, -1), gamma.reshape(1, -1),
      beta.reshape(1, -1), W2, b2.reshape(1, -1),
      routing_scores.reshape(1, -1))


def _sc_routing_call(logits):
    mesh = plsc.VectorSubcoreMesh(core_axis_name="c", subcore_axis_name="s")

    @functools.partial(
        pl.kernel,
        mesh=mesh,
        out_type=[
            jax.ShapeDtypeStruct((B, 16), jnp.float32),
            jax.ShapeDtypeStruct((16,), jnp.int32),
        ],
        scratch_types=[
            pltpu.VMEM((NUM_EXPERTS,), jnp.float32),
            pltpu.VMEM((16,), jnp.float32),
            pltpu.VMEM((16,), jnp.int32),
        ],
        compiler_params=pltpu.CompilerParams(needs_layout_passes=False),
    )
    def sc_routing(logits_hbm, wts_hbm, sel_hbm, lg_v, w_v, sel_v):
        cid = lax.axis_index("c")
        sid = lax.axis_index("s")
        wid = sid * 2 + cid

        @pl.when(wid < B)
        def _():
            b = wid
            pltpu.sync_copy(logits_hbm.at[b], lg_v)
            lane = lax.iota(jnp.int32, 16)
            lg = [lg_v[pl.ds(i * 16, 16)] for i in range(NVREG)]
            acts = [jnp.maximum(v - THRESHOLD, 0.0) for v in lg]
            for _ in range(3):
                tot = acts[0].sum()
                for i in range(1, NVREG):
                    tot = tot + acts[i].sum()
                acts = [
                    jnp.maximum(lg[i] - THRESHOLD - LATERAL * (tot - acts[i]),
                                0.0)
                    for i in range(NVREG)
                ]
            work = list(acts)
            idx = [lane + 16 * i for i in range(NVREG)]
            vals = jnp.zeros((16,), jnp.float32)
            sel = jnp.zeros((16,), jnp.int32)
            for k in range(TOP_K):
                m = work[0].max()
                for i in range(1, NVREG):
                    m = jnp.maximum(m, work[i].max())
                first = jnp.int32(NUM_EXPERTS)
                for i in range(NVREG):
                    cand = jnp.where(work[i] == m, idx[i],
                                     NUM_EXPERTS).min()
                    first = jnp.minimum(first, cand)
                vals = jnp.where(lane == k, m, vals)
                sel = jnp.where(lane == k, first, sel)
                work = [
                    jnp.where(idx[i] == first, -BIG, work[i])
                    for i in range(NVREG)
                ]
            ink = lane < TOP_K
            vmax = jnp.where(ink, vals, -BIG).max()
            ex = jnp.where(ink, jnp.exp(vals - vmax), 0.0)
            w = ex / ex.sum()
            w_v[...] = w
            pltpu.sync_copy(w_v, wts_hbm.at[b])

            @pl.when(b == 0)
            def _sel0():
                sel_v[...] = sel
                pltpu.sync_copy(sel_v, sel_hbm)

    return sc_routing(logits)


@jax.jit
def _run(hidden_states, W1, b1, gamma, beta, W2, b2, routing_scores):
    features, logits = _tc_dense(hidden_states, W1, b1, gamma, beta, W2, b2,
                                 routing_scores)
    wts16, sel16 = _sc_routing_call(logits)
    return features, sel16[:TOP_K], wts16[:, :TOP_K]


def kernel(hidden_states, W1, b1, gamma, beta, W2, b2, routing_scores):
    return _run(hidden_states, W1, b1, gamma, beta, W2, b2, routing_scores)


# speedup vs baseline: 1.2833x; 1.2833x over previous
import jax
import jax.numpy as jnp
from jax.experimental import pallas as pl
from jax.experimental.pallas import tpu as pltpu

B, S, D_MODEL = 4, 2048, 4096
ROUTER = 1024
NUM_EXPERTS = 64
TOP_K = 8
THRESHOLD = 0.1
LATERAL = 0.1

DCHUNK = 1024
SCHUNK = 512
NSTEPS = D_MODEL // DCHUNK
NSSTEPS = S // (2 * SCHUNK)


def _fused_body(hida_ref, hidb_ref, w1_ref, b1_ref, g_ref, be_ref, w2_ref,
                b2_ref, rs_ref, feat_ref, sel_ref, wts_ref, h_ref, pool_ref):
    i = pl.program_id(0)
    j = pl.program_id(1)

    part = jnp.sum(hida_ref[...], axis=1) + jnp.sum(hidb_ref[...], axis=1)

    @pl.when(j == 0)
    def _pinit():
        pool_ref[...] = part

    @pl.when(j > 0)
    def _pacc():
        pool_ref[...] += part

    @pl.when(j == NSSTEPS - 1)
    def _proj():
        pooled_j = pool_ref[...] * (1.0 / S)
        hj = jax.lax.dot_general(
            pooled_j, w1_ref[...], (((1,), (1,)), ((), ())),
            preferred_element_type=jnp.float32)

        @pl.when(i == 0)
        def _init():
            h_ref[...] = hj

        @pl.when(i > 0)
        def _acc():
            h_ref[...] += hj

    @pl.when((i == NSTEPS - 1) & (j == NSSTEPS - 1))
    def _head():
        h = h_ref[...] + b1_ref[...]
        mu = jnp.mean(h, axis=-1, keepdims=True)
        var = jnp.mean((h - mu) ** 2, axis=-1, keepdims=True)
        h = (h - mu) / jnp.sqrt(var + 1e-5) * g_ref[...] + be_ref[...]
        h = 0.5 * h * (1.0 + jax.lax.erf(h * (2.0 ** -0.5)))
        features = jax.lax.dot_general(
            h, w2_ref[...], (((1,), (1,)), ((), ())),
            preferred_element_type=jnp.float32) + b2_ref[...]
        feat_ref[...] = features

        affinity = jax.lax.dot_general(
            features, w2_ref[0:NUM_EXPERTS, :], (((1,), (1,)), ((), ())),
            preferred_element_type=jnp.float32)
        logits = rs_ref[...] + 0.1 * affinity

        acts = jnp.maximum(logits - THRESHOLD, 0.0)
        for _ in range(3):
            total = jnp.sum(acts, axis=-1, keepdims=True)
            inhibition = LATERAL * (total - acts)
            acts = jnp.maximum(logits - THRESHOLD - inhibition, 0.0)

        idx = jax.lax.broadcasted_iota(jnp.int32, (B, NUM_EXPERTS), 1)
        kidx = jax.lax.broadcasted_iota(jnp.int32, (B, TOP_K), 1)
        work = acts
        vals = jnp.zeros((B, TOP_K), jnp.float32)
        sel = jnp.zeros((B, TOP_K), jnp.int32)
        for k in range(TOP_K):
            m = jnp.max(work, axis=-1, keepdims=True)
            first = jnp.min(jnp.where(work == m, idx, NUM_EXPERTS),
                            axis=-1, keepdims=True)
            vals = jnp.where(kidx == k, m, vals)
            sel = jnp.where(kidx == k, first, sel)
            work = jnp.where(idx == first, -jnp.inf, work)
        wmax = jnp.max(vals, axis=-1, keepdims=True)
        ex = jnp.exp(vals - wmax)
        wts_ref[...] = ex / jnp.sum(ex, axis=-1, keepdims=True)
        sel_ref[...] = sel


@jax.jit
def _run(hidden_states, W1, b1, gamma, beta, W2, b2, routing_scores):
    features, sel, wts = pl.pallas_call(
        _fused_body,
        grid=(NSTEPS, NSSTEPS),
        in_specs=[
            pl.BlockSpec((B, SCHUNK, DCHUNK), lambda i, j: (0, j, i)),
            pl.BlockSpec((B, SCHUNK, DCHUNK),
                         lambda i, j: (0, j + NSSTEPS, i)),
            pl.BlockSpec((ROUTER, DCHUNK), lambda i, j: (0, i)),
            pl.BlockSpec((1, ROUTER), lambda i, j: (0, 0)),
            pl.BlockSpec((1, ROUTER), lambda i, j: (0, 0)),
            pl.BlockSpec((1, ROUTER), lambda i, j: (0, 0)),
            pl.BlockSpec((ROUTER, ROUTER), lambda i, j: (0, 0)),
            pl.BlockSpec((1, ROUTER), lambda i, j: (0, 0)),
            pl.BlockSpec((1, NUM_EXPERTS), lambda i, j: (0, 0)),
        ],
        out_specs=[
            pl.BlockSpec((B, ROUTER), lambda i, j: (0, 0)),
            pl.BlockSpec((B, TOP_K), lambda i, j: (0, 0)),
            pl.BlockSpec((B, TOP_K), lambda i, j: (0, 0)),
        ],
        out_shape=[
            jax.ShapeDtypeStruct((B, ROUTER), jnp.float32),
            jax.ShapeDtypeStruct((B, TOP_K), jnp.int32),
            jax.ShapeDtypeStruct((B, TOP_K), jnp.float32),
        ],
        scratch_shapes=[pltpu.VMEM((B, ROUTER), jnp.float32),
                        pltpu.VMEM((B, DCHUNK), jnp.float32)],
        compiler_params=pltpu.CompilerParams(
            dimension_semantics=("arbitrary", "arbitrary")),
    )(hidden_states, hidden_states, W1, b1.reshape(1, -1),
      gamma.reshape(1, -1), beta.reshape(1, -1), W2, b2.reshape(1, -1),
      routing_scores.reshape(1, -1))
    return features, sel[0], wts


def kernel(hidden_states, W1, b1, gamma, beta, W2, b2, routing_scores):
    return _run(hidden_states, W1, b1, gamma, beta, W2, b2, routing_scores)
